# baseline (speedup 1.0000x reference)
import functools

import jax
import jax.numpy as jnp
from jax import lax
from jax.experimental import pallas as pl
from jax.experimental.pallas import tpu as pltpu
from jax.experimental.pallas import tpu_sc as plsc


def _matmul_body(c_ref, w_ref, o_ref):
    o_ref[...] = jnp.dot(c_ref[...], w_ref[...],
                         preferred_element_type=jnp.float32)


def _basis_matmul(coeff, W):
    num_rels, _ = coeff.shape
    hidden = W.shape[1]
    return pl.pallas_call(
        _matmul_body,
        out_shape=jax.ShapeDtypeStruct((num_rels, hidden), jnp.float32),
    )(coeff, W)


@functools.cache
def _make_gather(E, D, V, NC, NS, chunk, nchunks):
    NW = NC * NS
    per = E // NW
    mesh = plsc.VectorSubcoreMesh(core_axis_name="c", subcore_axis_name="s")

    NBUF = 4
    PRE = 2

    @functools.partial(
        pl.kernel,
        mesh=mesh,
        compiler_params=pltpu.CompilerParams(use_tc_tiling_on_sc=False),
        out_type=jax.ShapeDtypeStruct((E, D), jnp.float32),
        scratch_types=[
            pltpu.VMEM((per,), jnp.int32),
            pltpu.VMEM_SHARED((V, D), jnp.float32),
        ] + [pltpu.VMEM((chunk, D), jnp.float32) for _ in range(NBUF)]
          + [pltpu.SemaphoreType.DMA for _ in range(2 * NBUF)],
    )
    def gather_kernel(idx_hbm, table_hbm, out_hbm, idx_v, tab_s,
                      *bufs_sems):
        bufs = bufs_sems[:NBUF]
        gsem = bufs_sems[NBUF:2 * NBUF]
        wsem = bufs_sems[2 * NBUF:]
        wid = lax.axis_index("s") * NC + lax.axis_index("c")
        base = wid * per

        @pl.when(lax.axis_index("s") == 0)
        def _():
            pltpu.sync_copy(table_hbm, tab_s)
        plsc.subcore_barrier()

        pltpu.sync_copy(idx_hbm.at[pl.ds(base, per)], idx_v)

        def fire_gather(j, b):
            pltpu.async_copy(
                tab_s.at[idx_v.at[pl.ds(j * chunk, chunk)]],
                bufs[b], gsem[b])

        for j in range(min(PRE, nchunks)):
            fire_gather(j, j % NBUF)

        def body(j, carry):
            for i in range(NBUF):
                @pl.when(j % NBUF == i)
                def _():
                    bn = (i + PRE) % NBUF

                    @pl.when(j + PRE < nchunks)
                    def _():
                        @pl.when(j + PRE >= NBUF)
                        def _():
                            pltpu.make_async_copy(
                                bufs[bn],
                                out_hbm.at[pl.ds(base, chunk)],
                                wsem[bn]).wait()
                        fire_gather(j + PRE, bn)

                    pltpu.make_async_copy(
                        tab_s.at[idx_v.at[pl.ds(0, chunk)]],
                        bufs[i], gsem[i]).wait()
                    pltpu.async_copy(
                        bufs[i],
                        out_hbm.at[pl.ds(base + j * chunk, chunk)],
                        wsem[i])
            return carry

        lax.fori_loop(0, nchunks, body, 0)

        for i in range(min(NBUF, nchunks)):
            pltpu.make_async_copy(
                bufs[i], out_hbm.at[pl.ds(base, chunk)], wsem[i]).wait()

    return gather_kernel


def kernel(etype, coeff, W):
    E = etype.shape[0]
    D = W.shape[1]
    info = plsc.get_sparse_core_info()
    NC, NS = info.num_cores, info.num_subcores
    NW = NC * NS
    per = E // NW
    assert per * NW == E
    chunk = 8
    for c in range(8, 257, 8):
        if per % c == 0:
            chunk = c
    nchunks = per // chunk

    w = _basis_matmul(coeff, W)
    idx = etype.astype(jnp.int32)
    out = _make_gather(E, D, coeff.shape[0], NC, NS, chunk, nchunks)(idx, w)
    return out + 1.0


# speedup vs baseline: 3.0493x; 3.0493x over previous
import functools

import jax
import jax.numpy as jnp
from jax import lax
from jax.experimental import pallas as pl
from jax.experimental.pallas import tpu as pltpu
from jax.experimental.pallas import tpu_sc as plsc


def _matmul_body(c_ref, w_ref, o_ref):
    o_ref[...] = jnp.dot(c_ref[...], w_ref[...],
                         preferred_element_type=jnp.float32)


def _basis_matmul(coeff, W):
    num_rels, _ = coeff.shape
    hidden = W.shape[1]
    return pl.pallas_call(
        _matmul_body,
        out_shape=jax.ShapeDtypeStruct((num_rels, hidden), jnp.float32),
    )(coeff, W)


@functools.cache
def _make_gather(E, D, V, NC, NS, chunk, nchunks):
    NW = NC * NS
    per = E // NW
    mesh = plsc.VectorSubcoreMesh(core_axis_name="c", subcore_axis_name="s")

    NBUF = 4
    PRE = 2

    @functools.partial(
        pl.kernel,
        mesh=mesh,
        compiler_params=pltpu.CompilerParams(use_tc_tiling_on_sc=False),
        out_type=jax.ShapeDtypeStruct((E, 2 * D), jnp.float32),
        scratch_types=[
            pltpu.VMEM((per,), jnp.int32),
            pltpu.VMEM_SHARED((V, D), jnp.float32),
        ] + [pltpu.VMEM((chunk, D), jnp.float32) for _ in range(NBUF)]
          + [pltpu.SemaphoreType.DMA for _ in range(2 * NBUF)],
    )
    def gather_kernel(idx_hbm, table_hbm, out_hbm, idx_v, tab_s,
                      *bufs_sems):
        bufs = bufs_sems[:NBUF]
        gsem = bufs_sems[NBUF:2 * NBUF]
        wsem = bufs_sems[2 * NBUF:]
        wid = lax.axis_index("s") * NC + lax.axis_index("c")
        base = wid * per

        @pl.when(lax.axis_index("s") == 0)
        def _():
            pltpu.sync_copy(table_hbm, tab_s)
        plsc.subcore_barrier()

        pltpu.sync_copy(idx_hbm.at[pl.ds(base, per)], idx_v)

        def fire_gather(j, b):
            pltpu.async_copy(
                tab_s.at[idx_v.at[pl.ds(j * chunk, chunk)]],
                bufs[b], gsem[b])

        for j in range(min(PRE, nchunks)):
            fire_gather(j, j % NBUF)

        def body(j, carry):
            for i in range(NBUF):
                @pl.when(j % NBUF == i)
                def _():
                    bn = (i + PRE) % NBUF

                    @pl.when(j + PRE < nchunks)
                    def _():
                        @pl.when(j + PRE >= NBUF)
                        def _():
                            pltpu.make_async_copy(
                                bufs[bn],
                                out_hbm.at[pl.ds(base, chunk), pl.ds(0, D)],
                                wsem[bn]).wait()
                        fire_gather(j + PRE, bn)

                    pltpu.make_async_copy(
                        tab_s.at[idx_v.at[pl.ds(0, chunk)]],
                        bufs[i], gsem[i]).wait()
                    pltpu.async_copy(
                        bufs[i],
                        out_hbm.at[pl.ds(base + j * chunk, chunk),
                                   pl.ds(0, D)],
                        wsem[i])
            return carry

        lax.fori_loop(0, nchunks, body, 0)

        for i in range(min(NBUF, nchunks)):
            pltpu.make_async_copy(
                bufs[i], out_hbm.at[pl.ds(base, chunk), pl.ds(0, D)],
                wsem[i]).wait()

    return gather_kernel


def kernel(etype, coeff, W):
    E = etype.shape[0]
    D = W.shape[1]
    info = plsc.get_sparse_core_info()
    NC, NS = info.num_cores, info.num_subcores
    NW = NC * NS
    per = E // NW
    assert per * NW == E
    chunk = 8
    for c in range(8, 257, 8):
        if per % c == 0:
            chunk = c
    nchunks = per // chunk

    w = _basis_matmul(coeff, W)
    idx = etype.astype(jnp.int32)
    out = _make_gather(E, D, coeff.shape[0], NC, NS, chunk, nchunks)(idx, w)
    return out[:, :D]


# speedup vs baseline: 3.0532x; 1.0013x over previous
import functools

import jax
import jax.numpy as jnp
from jax import lax
from jax.experimental import pallas as pl
from jax.experimental.pallas import tpu as pltpu
from jax.experimental.pallas import tpu_sc as plsc


def _matmul_body(c_ref, w_ref, o_ref):
    o_ref[...] = jnp.dot(c_ref[...], w_ref[...],
                         preferred_element_type=jnp.float32)


def _basis_matmul(coeff, W):
    num_rels, _ = coeff.shape
    hidden = W.shape[1]
    return pl.pallas_call(
        _matmul_body,
        out_shape=jax.ShapeDtypeStruct((num_rels, hidden), jnp.float32),
    )(coeff, W)


@functools.cache
def _make_gather(E, D, V, NC, NS, chunk, nchunks):
    NW = NC * NS
    per = E // NW
    mesh = plsc.VectorSubcoreMesh(core_axis_name="c", subcore_axis_name="s")

    NBUF = 6
    PRE = 3

    @functools.partial(
        pl.kernel,
        mesh=mesh,
        compiler_params=pltpu.CompilerParams(use_tc_tiling_on_sc=False),
        out_type=jax.ShapeDtypeStruct((E, 2 * D), jnp.float32),
        scratch_types=[
            pltpu.VMEM((per,), jnp.int32),
            pltpu.VMEM_SHARED((V, D), jnp.float32),
        ] + [pltpu.VMEM((chunk, D), jnp.float32) for _ in range(NBUF)]
          + [pltpu.SemaphoreType.DMA for _ in range(2 * NBUF)],
    )
    def gather_kernel(idx_hbm, table_hbm, out_hbm, idx_v, tab_s,
                      *bufs_sems):
        bufs = bufs_sems[:NBUF]
        gsem = bufs_sems[NBUF:2 * NBUF]
        wsem = bufs_sems[2 * NBUF:]
        wid = lax.axis_index("s") * NC + lax.axis_index("c")
        base = wid * per

        @pl.when(lax.axis_index("s") == 0)
        def _():
            pltpu.sync_copy(table_hbm, tab_s)
        plsc.subcore_barrier()

        pltpu.sync_copy(idx_hbm.at[pl.ds(base, per)], idx_v)

        def fire_gather(j, b):
            pltpu.async_copy(
                tab_s.at[idx_v.at[pl.ds(j * chunk, chunk)]],
                bufs[b], gsem[b])

        for j in range(min(PRE, nchunks)):
            fire_gather(j, j % NBUF)

        def body(j, carry):
            for i in range(NBUF):
                @pl.when(j % NBUF == i)
                def _():
                    bn = (i + PRE) % NBUF

                    @pl.when(j + PRE < nchunks)
                    def _():
                        @pl.when(j + PRE >= NBUF)
                        def _():
                            pltpu.make_async_copy(
                                bufs[bn],
                                out_hbm.at[pl.ds(base, chunk), pl.ds(0, D)],
                                wsem[bn]).wait()
                        fire_gather(j + PRE, bn)

                    pltpu.make_async_copy(
                        tab_s.at[idx_v.at[pl.ds(0, chunk)]],
                        bufs[i], gsem[i]).wait()
                    pltpu.async_copy(
                        bufs[i],
                        out_hbm.at[pl.ds(base + j * chunk, chunk),
                                   pl.ds(0, D)],
                        wsem[i])
            return carry

        lax.fori_loop(0, nchunks, body, 0)

        for i in range(min(NBUF, nchunks)):
            pltpu.make_async_copy(
                bufs[i], out_hbm.at[pl.ds(base, chunk), pl.ds(0, D)],
                wsem[i]).wait()

    return gather_kernel


def kernel(etype, coeff, W):
    E = etype.shape[0]
    D = W.shape[1]
    info = plsc.get_sparse_core_info()
    NC, NS = info.num_cores, info.num_subcores
    NW = NC * NS
    per = E // NW
    assert per * NW == E
    chunk = 8
    for c in range(8, 401, 8):
        if per % c == 0:
            chunk = c
    nchunks = per // chunk

    w = _basis_matmul(coeff, W)
    idx = etype.astype(jnp.int32)
    out = _make_gather(E, D, coeff.shape[0], NC, NS, chunk, nchunks)(idx, w)
    return out[:, :D]
